# Initial kernel scaffold; baseline (speedup 1.0000x reference)
import jax
import jax.numpy as jnp
from jax.experimental import pallas as pl


def kernel(node_energy, local_or_ghost, batch, ptr, positions, cell, forces):
    raise NotImplementedError("write your pallas kernel here")


# speedup vs baseline: 4.3017x; 4.3017x over previous
import functools

import jax
import jax.numpy as jnp
from jax import lax
from jax.experimental import pallas as pl
from jax.experimental.pallas import tpu as pltpu
from jax.experimental.pallas import tpu_sc as plsc

G = 16
L = 16
NUM_TILES = 16
CHUNK = 6256
NPAD = NUM_TILES * CHUNK


def _seg_sum_body(ne_hbm, lg_hbm, b_hbm, out_hbm, stage_hbm, ne_v, lg_v, b_v,
                  part_v, all_v, stage_v):
    sid = lax.axis_index("s")
    base = sid * CHUNK
    pltpu.sync_copy(ne_hbm.at[pl.ds(base, CHUNK)], ne_v)
    pltpu.sync_copy(lg_hbm.at[pl.ds(base, CHUNK)], lg_v)
    pltpu.sync_copy(b_hbm.at[pl.ds(base, CHUNK)], b_v)

    zero = jnp.zeros((L,), jnp.float32)

    def body(i, accs):
        off = i * L
        e = ne_v[pl.ds(off, L)] * lg_v[pl.ds(off, L)]
        b = b_v[pl.ds(off, L)]
        return tuple(acc + jnp.where(b == g, e, zero)
                     for g, acc in enumerate(accs))

    accs = lax.fori_loop(0, CHUNK // L, body,
                         tuple(zero for _ in range(G)))

    for g in range(G):
        all_v[g] = accs[g]
    rows = lax.iota(jnp.int32, L)
    total = zero
    for l in range(L):
        total = total + plsc.load_gather(
            all_v, [rows, jnp.full((L,), l, jnp.int32)])
    part_v[...] = total

    pltpu.sync_copy(part_v, stage_hbm.at[pl.ds(sid * L, L)])
    plsc.subcore_barrier()

    @pl.when(sid == 0)
    def _():
        pltpu.sync_copy(stage_hbm, stage_v)
        tot = stage_v[pl.ds(0, L)]
        for t in range(1, NUM_TILES):
            tot = tot + stage_v[pl.ds(t * L, L)]
        part_v[...] = tot
        pltpu.sync_copy(part_v, out_hbm)


@functools.partial(jax.jit, static_argnames=())
def _segment_energy(ne, lg, b):
    mesh = plsc.VectorSubcoreMesh(core_axis_name="c", subcore_axis_name="s",
                                  num_cores=1)
    f = pl.kernel(
        _seg_sum_body,
        out_type=[
            jax.ShapeDtypeStruct((G,), jnp.float32),
            jax.ShapeDtypeStruct((NUM_TILES * L,), jnp.float32),
        ],
        mesh=mesh,
        scratch_types=[
            pltpu.VMEM((CHUNK,), jnp.float32),
            pltpu.VMEM((CHUNK,), jnp.float32),
            pltpu.VMEM((CHUNK,), jnp.int32),
            pltpu.VMEM((L,), jnp.float32),
            pltpu.VMEM((G, L), jnp.float32),
            pltpu.VMEM((NUM_TILES * L,), jnp.float32),
        ],
        compiler_params=pltpu.CompilerParams(needs_layout_passes=False),
    )
    return f(ne, lg, b)[0]


def kernel(node_energy, local_or_ghost, batch, ptr, positions, cell, forces):
    n = node_energy.shape[0]
    pad = NPAD - n
    ne = jnp.pad(node_energy, (0, pad))
    lg = jnp.pad(local_or_ghost, (0, pad))
    b = jnp.pad(batch.astype(jnp.int32), (0, pad))
    total_energy_local = _segment_energy(ne, lg, b)
    virials = jnp.zeros_like(cell)
    return (total_energy_local, node_energy, forces, virials)


# speedup vs baseline: 4.5018x; 1.0465x over previous
import functools

import jax
import jax.numpy as jnp
from jax import lax
from jax.experimental import pallas as pl
from jax.experimental.pallas import tpu as pltpu
from jax.experimental.pallas import tpu_sc as plsc

G = 16
L = 16
NUM_TILES = 16
N = 100000
CHUNK = 6256
SKIP15 = (NUM_TILES * CHUNK - N) // L


def _seg_sum_body(ne_hbm, lg_hbm, b_hbm, out_hbm, stage_hbm, ne_v, lg_v, b_v,
                  part_v, all_v, stage_v):
    sid = lax.axis_index("s")
    last = sid == NUM_TILES - 1
    base = jnp.where(last, N - CHUNK, sid * CHUNK)
    pltpu.sync_copy(ne_hbm.at[pl.ds(base, CHUNK)], ne_v)
    pltpu.sync_copy(lg_hbm.at[pl.ds(base, CHUNK)], lg_v)
    pltpu.sync_copy(b_hbm.at[pl.ds(base, CHUNK)], b_v)

    zero = jnp.zeros((L,), jnp.float32)

    def body(i, accs):
        off = i * L
        e = ne_v[pl.ds(off, L)] * lg_v[pl.ds(off, L)]
        b = b_v[pl.ds(off, L)]
        return tuple(acc + jnp.where(b == g, e, zero)
                     for g, acc in enumerate(accs))

    lb = jnp.where(last, SKIP15, 0)
    accs = lax.fori_loop(lb, CHUNK // L, body,
                         tuple(zero for _ in range(G)))

    for g in range(G):
        all_v[g] = accs[g]
    rows = lax.iota(jnp.int32, L)
    total = zero
    for l in range(L):
        total = total + plsc.load_gather(
            all_v, [rows, jnp.full((L,), l, jnp.int32)])
    part_v[...] = total

    pltpu.sync_copy(part_v, stage_hbm.at[pl.ds(sid * L, L)])
    plsc.subcore_barrier()

    @pl.when(sid == 0)
    def _():
        pltpu.sync_copy(stage_hbm, stage_v)
        tot = stage_v[pl.ds(0, L)]
        for t in range(1, NUM_TILES):
            tot = tot + stage_v[pl.ds(t * L, L)]
        part_v[...] = tot
        pltpu.sync_copy(part_v, out_hbm)


@functools.partial(jax.jit, static_argnames=())
def _segment_energy(ne, lg, b):
    mesh = plsc.VectorSubcoreMesh(core_axis_name="c", subcore_axis_name="s",
                                  num_cores=1)
    f = pl.kernel(
        _seg_sum_body,
        out_type=[
            jax.ShapeDtypeStruct((G,), jnp.float32),
            jax.ShapeDtypeStruct((NUM_TILES * L,), jnp.float32),
        ],
        mesh=mesh,
        scratch_types=[
            pltpu.VMEM((CHUNK,), jnp.float32),
            pltpu.VMEM((CHUNK,), jnp.float32),
            pltpu.VMEM((CHUNK,), jnp.int32),
            pltpu.VMEM((L,), jnp.float32),
            pltpu.VMEM((G, L), jnp.float32),
            pltpu.VMEM((NUM_TILES * L,), jnp.float32),
        ],
        compiler_params=pltpu.CompilerParams(needs_layout_passes=False),
    )
    return f(ne, lg, b)[0]


def kernel(node_energy, local_or_ghost, batch, ptr, positions, cell, forces):
    total_energy_local = _segment_energy(node_energy, local_or_ghost,
                                         batch.astype(jnp.int32))
    virials = jnp.zeros_like(cell)
    return (total_energy_local, node_energy, forces, virials)


# speedup vs baseline: 4.7649x; 1.0585x over previous
import functools

import jax
import jax.numpy as jnp
from jax import lax
from jax.experimental import pallas as pl
from jax.experimental.pallas import tpu as pltpu
from jax.experimental.pallas import tpu_sc as plsc

G = 16
L = 16
NUM_TILES = 16
NUM_CORES = 2
SEGS = G // NUM_CORES
N = 100000
CHUNK = 6256
NVEC = CHUNK // L
SKIP15 = (NUM_TILES * CHUNK - N) // L


def _seg_sum_body(ne_hbm, lg_hbm, b_hbm, out_hbm, stage_hbm, ne_v, lg_v, b_v,
                  part_v, all_v, stage_v, sem1, sem2, sem3):
    cid = lax.axis_index("c")
    sid = lax.axis_index("s")
    last = sid == NUM_TILES - 1
    base = jnp.where(last, N - CHUNK, sid * CHUNK)
    c1 = pltpu.async_copy(ne_hbm.at[pl.ds(base, CHUNK)], ne_v, sem1)
    c2 = pltpu.async_copy(lg_hbm.at[pl.ds(base, CHUNK)], lg_v, sem2)
    c3 = pltpu.async_copy(b_hbm.at[pl.ds(base, CHUNK)], b_v, sem3)
    c1.wait()
    c2.wait()
    c3.wait()

    zero = jnp.zeros((L,), jnp.float32)
    seg0 = cid * SEGS

    def acc_one(off, accs):
        e = ne_v[pl.ds(off, L)] * lg_v[pl.ds(off, L)]
        b = b_v[pl.ds(off, L)]
        return tuple(acc + jnp.where(b == seg0 + g, e, zero)
                     for g, acc in enumerate(accs))

    def body(i, accs):
        accs = acc_one(i * (2 * L), accs)
        return acc_one(i * (2 * L) + L, accs)

    lb = jnp.where(last, SKIP15 // 2, 0)
    accs = lax.fori_loop(lb, NVEC // 2, body,
                         tuple(zero for _ in range(SEGS)))
    accs = acc_one((NVEC - 1) * L, accs)

    for g in range(SEGS):
        all_v[g] = accs[g]
    rows = jnp.bitwise_and(lax.iota(jnp.int32, L), SEGS - 1)
    total = zero
    for l in range(L):
        total = total + plsc.load_gather(
            all_v, [rows, jnp.full((L,), l, jnp.int32)])
    part_v[...] = total

    srow = (cid * NUM_TILES + sid) * L
    pltpu.sync_copy(part_v, stage_hbm.at[pl.ds(srow, L)])
    plsc.subcore_barrier()

    @pl.when(sid == 0)
    def _():
        pltpu.sync_copy(stage_hbm.at[pl.ds(cid * NUM_TILES * L,
                                           NUM_TILES * L)], stage_v)
        tot = stage_v[pl.ds(0, L)]
        for t in range(1, NUM_TILES):
            tot = tot + stage_v[pl.ds(t * L, L)]
        part_v[...] = tot
        pltpu.sync_copy(part_v.at[pl.ds(0, SEGS)],
                        out_hbm.at[pl.ds(cid * SEGS, SEGS)])


@jax.jit
def _segment_energy(ne, lg, b):
    mesh = plsc.VectorSubcoreMesh(core_axis_name="c", subcore_axis_name="s")
    f = pl.kernel(
        _seg_sum_body,
        out_type=[
            jax.ShapeDtypeStruct((G,), jnp.float32),
            jax.ShapeDtypeStruct((NUM_CORES * NUM_TILES * L,), jnp.float32),
        ],
        mesh=mesh,
        scratch_types=[
            pltpu.VMEM((CHUNK,), jnp.float32),
            pltpu.VMEM((CHUNK,), jnp.float32),
            pltpu.VMEM((CHUNK,), jnp.int32),
            pltpu.VMEM((L,), jnp.float32),
            pltpu.VMEM((SEGS, L), jnp.float32),
            pltpu.VMEM((NUM_TILES * L,), jnp.float32),
            pltpu.SemaphoreType.DMA,
            pltpu.SemaphoreType.DMA,
            pltpu.SemaphoreType.DMA,
        ],
        compiler_params=pltpu.CompilerParams(needs_layout_passes=False),
    )
    return f(ne, lg, b)[0]


def kernel(node_energy, local_or_ghost, batch, ptr, positions, cell, forces):
    total_energy_local = _segment_energy(node_energy, local_or_ghost,
                                         batch.astype(jnp.int32))
    virials = jnp.zeros_like(cell)
    return (total_energy_local, node_energy, forces, virials)


# speedup vs baseline: 4.7679x; 1.0006x over previous
import functools

import jax
import jax.numpy as jnp
from jax import lax
from jax.experimental import pallas as pl
from jax.experimental.pallas import tpu as pltpu
from jax.experimental.pallas import tpu_sc as plsc

G = 16
L = 16
NUM_TILES = 16
NUM_CORES = 2
SEGS = G // NUM_CORES
N = 100000
CHUNK = 6256
NVEC = CHUNK // L
SKIP15 = (NUM_TILES * CHUNK - N) // L


def _seg_sum_body(ne_hbm, lg_hbm, b_hbm, out_hbm, stage_hbm, ne_v, lg_v, b_v,
                  part_v, all_v, stage_v, sem1, sem2, sem3):
    cid = lax.axis_index("c")
    sid = lax.axis_index("s")
    last = sid == NUM_TILES - 1
    base = jnp.where(last, N - CHUNK, sid * CHUNK)
    c1 = pltpu.async_copy(ne_hbm.at[pl.ds(base, CHUNK)], ne_v, sem1)
    c2 = pltpu.async_copy(lg_hbm.at[pl.ds(base, CHUNK)], lg_v, sem2)
    c3 = pltpu.async_copy(b_hbm.at[pl.ds(base, CHUNK)], b_v, sem3)
    c1.wait()
    c2.wait()
    c3.wait()

    zero = jnp.zeros((L,), jnp.float32)
    seg0 = cid * SEGS

    def acc_one(off, accs):
        e = ne_v[pl.ds(off, L)] * lg_v[pl.ds(off, L)]
        b = b_v[pl.ds(off, L)]
        return tuple(acc + jnp.where(b == seg0 + g, e, zero)
                     for g, acc in enumerate(accs))

    def body(i, accs):
        accs = acc_one(i * (2 * L), accs)
        return acc_one(i * (2 * L) + L, accs)

    lb = jnp.where(last, SKIP15 // 2, 0)
    accs = lax.fori_loop(lb, NVEC // 2, body,
                         tuple(zero for _ in range(SEGS)))
    accs = acc_one((NVEC - 1) * L, accs)

    for g in range(SEGS):
        all_v[g] = accs[g]
    rows = jnp.bitwise_and(lax.iota(jnp.int32, L), SEGS - 1)
    total = zero
    for l in range(L):
        total = total + plsc.load_gather(
            all_v, [rows, jnp.full((L,), l, jnp.int32)])
    part_v[...] = total

    srow = (cid * NUM_TILES + sid) * L
    pltpu.sync_copy(part_v, stage_hbm.at[pl.ds(srow, L)])
    plsc.subcore_barrier()

    @pl.when(sid == 0)
    def _():
        pltpu.sync_copy(stage_hbm.at[pl.ds(cid * NUM_TILES * L,
                                           NUM_TILES * L)], stage_v)
        tot = stage_v[pl.ds(0, L)]
        for t in range(1, NUM_TILES):
            tot = tot + stage_v[pl.ds(t * L, L)]
        part_v[...] = tot
        pltpu.sync_copy(part_v.at[pl.ds(0, SEGS)],
                        out_hbm.at[pl.ds(cid * SEGS, SEGS)])


@jax.jit
def _segment_energy(ne, lg, b):
    mesh = plsc.VectorSubcoreMesh(core_axis_name="c", subcore_axis_name="s")
    f = pl.kernel(
        _seg_sum_body,
        out_type=[
            jax.ShapeDtypeStruct((G,), jnp.float32),
            jax.ShapeDtypeStruct((NUM_CORES * NUM_TILES * L,), jnp.float32),
        ],
        mesh=mesh,
        scratch_types=[
            pltpu.VMEM((CHUNK,), jnp.float32),
            pltpu.VMEM((CHUNK,), jnp.float32),
            pltpu.VMEM((CHUNK,), jnp.int32),
            pltpu.VMEM((L,), jnp.float32),
            pltpu.VMEM((SEGS, L), jnp.float32),
            pltpu.VMEM((NUM_TILES * L,), jnp.float32),
            pltpu.SemaphoreType.DMA,
            pltpu.SemaphoreType.DMA,
            pltpu.SemaphoreType.DMA,
        ],
        compiler_params=pltpu.CompilerParams(needs_layout_passes=False, skip_device_barrier=True),
    )
    return f(ne, lg, b)[0]


def kernel(node_energy, local_or_ghost, batch, ptr, positions, cell, forces):
    total_energy_local = _segment_energy(node_energy, local_or_ghost,
                                         batch.astype(jnp.int32))
    virials = jnp.zeros_like(cell)
    return (total_energy_local, node_energy, forces, virials)
